# Initial kernel scaffold; baseline (speedup 1.0000x reference)
import jax
import jax.numpy as jnp
from jax.experimental import pallas as pl


def kernel(x, edge_index, hidden_state, gcn_W, gcn_b, W_ih, W_hh, b_ih, b_hh, fc_W, fc_b):
    raise NotImplementedError("write your pallas kernel here")


# speedup vs baseline: 8.6754x; 8.6754x over previous
import functools

import jax
import jax.numpy as jnp
from jax import lax
from jax.experimental import pallas as pl
from jax.experimental.pallas import tpu as pltpu
from jax.experimental.pallas import tpu_sc as plsc

HID = 64
G3 = 3 * HID


def _sc_deg_body(ew, ch, np_, dst_hbm, z_hbm, out_hbm, dstv, deg):
    nc = 2
    wid = lax.axis_index("s") * nc + lax.axis_index("c")
    base = wid * ew
    pltpu.sync_copy(z_hbm, deg)
    ones = jnp.full((16,), 1.0, jnp.float32)

    def chunk(i, _):
        pltpu.sync_copy(dst_hbm.at[pl.ds(base + i * ch, ch)], dstv)

        def inner(j, _):
            idx = dstv[pl.ds(j * 16, 16)]
            plsc.addupdate_scatter(deg, [idx], ones)
            return 0

        return lax.fori_loop(0, ch // 16, inner, 0)

    lax.fori_loop(0, ew // ch, chunk, 0)
    pltpu.sync_copy(deg, out_hbm.at[wid])


def _sc_deg(dst_pad, zeros1, np_, e_pad):
    ew = e_pad // 32
    ch = 1600
    mesh = plsc.VectorSubcoreMesh(core_axis_name="c", subcore_axis_name="s")
    f = pl.kernel(
        functools.partial(_sc_deg_body, ew, ch, np_),
        out_type=jax.ShapeDtypeStruct((32, np_), jnp.float32),
        mesh=mesh,
        scratch_types=[
            pltpu.VMEM((ch,), jnp.int32),
            pltpu.VMEM((np_,), jnp.float32),
        ],
        compiler_params=pltpu.CompilerParams(
            needs_layout_passes=False, use_tc_tiling_on_sc=False),
    )
    return f(dst_pad, zeros1)


def _sc_agg_body(np_, e_pad, srcx_hbm, dst2_hbm, hpc_hbm, zr_hbm, out_hbm,
                 srcv, dstv, rows, aggs, sem):
    c = lax.axis_index("c")
    s = lax.axis_index("s")
    rpt = np_ // 16
    pltpu.sync_copy(zr_hbm.at[pl.ds(s * rpt, rpt)], aggs.at[pl.ds(s * rpt, rpt)])
    plsc.subcore_barrier()

    erows = e_pad // 128
    rpt_e = erows // 16
    ebase = s * rpt_e
    cc = 4

    def chunk(i, _):
        off = ebase + i * cc
        pltpu.sync_copy(srcx_hbm.at[pl.ds(c * erows + off, cc)], srcv)
        pltpu.sync_copy(dst2_hbm.at[pl.ds(off, cc)], dstv)
        cops = []
        for j in range(cc):
            cops.append(pltpu.async_copy(
                hpc_hbm.at[srcv.at[j]], rows.at[pl.ds(j * 128, 128)], sem))
        for d in cops:
            d.wait()
        for j in range(cc):
            pltpu.sync_copy(rows.at[pl.ds(j * 128, 128)], aggs.at[dstv.at[j]], add=True)
        return 0

    lax.fori_loop(0, rpt_e // cc, chunk, 0)
    plsc.subcore_barrier()
    pltpu.sync_copy(aggs.at[pl.ds(s * rpt, rpt)], out_hbm.at[c].at[pl.ds(s * rpt, rpt)])


def _sc_agg(srcx, dst2, hpc, zrows, np_, e_pad):
    mesh = plsc.VectorSubcoreMesh(core_axis_name="c", subcore_axis_name="s")
    f = pl.kernel(
        functools.partial(_sc_agg_body, np_, e_pad),
        out_type=jax.ShapeDtypeStruct((2, np_, 32), jnp.float32),
        mesh=mesh,
        scratch_types=[
            pltpu.VMEM((4, 128), jnp.int32),
            pltpu.VMEM((4, 128), jnp.int32),
            pltpu.VMEM((512, 32), jnp.float32),
            pltpu.VMEM_SHARED((np_, 32), jnp.float32),
            pltpu.SemaphoreType.DMA,
        ],
        compiler_params=pltpu.CompilerParams(
            needs_layout_passes=False, use_tc_tiling_on_sc=False),
    )
    return f(srcx, dst2, hpc, zrows)


def _tc_prep_a_body(degp_ref, x_ref, w_ref, h_ref, hp_ref, dinv_ref):
    deg = jnp.sum(degp_ref[...], axis=0) + 1.0
    dinv = lax.rsqrt(deg).reshape(-1, 1)
    h = jnp.dot(x_ref[...], w_ref[...], preferred_element_type=jnp.float32)
    h_ref[...] = h
    hp_ref[...] = h * dinv
    dinv_ref[...] = dinv


def _tc_prep_a(degp, xp, gcn_W, np_):
    bn = 1024
    grid = np_ // bn
    return pl.pallas_call(
        _tc_prep_a_body,
        grid=(grid,),
        in_specs=[
            pl.BlockSpec((32, bn), lambda i: (0, i)),
            pl.BlockSpec((bn, 7), lambda i: (i, 0)),
            pl.BlockSpec((7, HID), lambda i: (0, 0)),
        ],
        out_specs=[
            pl.BlockSpec((bn, HID), lambda i: (i, 0)),
            pl.BlockSpec((bn, HID), lambda i: (i, 0)),
            pl.BlockSpec((bn, 1), lambda i: (i, 0)),
        ],
        out_shape=[
            jax.ShapeDtypeStruct((np_, HID), jnp.float32),
            jax.ShapeDtypeStruct((np_, HID), jnp.float32),
            jax.ShapeDtypeStruct((np_, 1), jnp.float32),
        ],
    )(degp, xp, gcn_W)


def _tc_prep_b_body(aggp_ref, dinv_ref, h_ref, gb_ref, wt_ref, bg_ref, gi_ref):
    dinv = dinv_ref[...]
    cat = jnp.concatenate([aggp_ref[0], aggp_ref[1]], axis=1)
    agg = cat * dinv + h_ref[...] * (dinv * dinv) + gb_ref[...]
    gi_ref[...] = (
        jnp.dot(agg, wt_ref[...], preferred_element_type=jnp.float32)
        + bg_ref[...]
    )


def _tc_prep_b(aggp, dinvc, h, gcn_b, w_ih_t, bias_g, n, np_):
    bn = 1000
    grid = n // bn
    return pl.pallas_call(
        _tc_prep_b_body,
        grid=(grid,),
        in_specs=[
            pl.BlockSpec((2, bn, 32), lambda i: (0, i, 0)),
            pl.BlockSpec((bn, 1), lambda i: (i, 0)),
            pl.BlockSpec((bn, HID), lambda i: (i, 0)),
            pl.BlockSpec((1, HID), lambda i: (0, 0)),
            pl.BlockSpec((HID, G3), lambda i: (0, 0)),
            pl.BlockSpec((1, G3), lambda i: (0, 0)),
        ],
        out_specs=pl.BlockSpec((bn, G3), lambda i: (i, 0)),
        out_shape=jax.ShapeDtypeStruct((n, G3), jnp.float32),
    )(aggp, dinvc, h, gcn_b, w_ih_t, bias_g)


def _tc_gru_body(bn, nblk, gi_ref, wt_ref, bn_ref, fw_ref, fb_ref, h0_ref,
                 y_ref, hl_ref, h_scr, outs_scr):
    pid = pl.program_id(0)

    @pl.when(pid == 0)
    def _init():
        h_scr[...] = h0_ref[...]

    wt = wt_ref[...]
    bvec = bn_ref[...]

    def step(t, h):
        gi = gi_ref[pl.ds(t, 1), :]
        gh = jnp.dot(h, wt, preferred_element_type=jnp.float32) + bvec
        a = gi + gh
        rz = jax.nn.sigmoid(a[:, :2 * HID])
        r = rz[:, :HID]
        z = rz[:, HID:]
        ng = jnp.tanh(gi[:, 2 * HID:] + r * gh[:, 2 * HID:])
        hn = z * (h - ng) + ng
        outs_scr[pl.ds(t, 1), :] = hn
        return hn

    h = lax.fori_loop(0, bn, step, h_scr[...])
    h_scr[...] = h
    y_ref[...] = (
        jnp.dot(outs_scr[...], fw_ref[...], preferred_element_type=jnp.float32)
        + fb_ref[...]
    )

    @pl.when(pid == nblk - 1)
    def _fin():
        hl_ref[...] = h


def _tc_gru(gi, w_hh_t, b_hh_n, fc_W, fc_b, h0, n):
    bn = 1000
    nblk = n // bn
    return pl.pallas_call(
        functools.partial(_tc_gru_body, bn, nblk),
        grid=(nblk,),
        in_specs=[
            pl.BlockSpec((bn, G3), lambda i: (i, 0)),
            pl.BlockSpec((HID, G3), lambda i: (0, 0)),
            pl.BlockSpec((1, G3), lambda i: (0, 0)),
            pl.BlockSpec((HID, 4), lambda i: (0, 0)),
            pl.BlockSpec((1, 4), lambda i: (0, 0)),
            pl.BlockSpec((1, HID), lambda i: (0, 0)),
        ],
        out_specs=[
            pl.BlockSpec((bn, 4), lambda i: (i, 0)),
            pl.BlockSpec((1, HID), lambda i: (0, 0)),
        ],
        out_shape=[
            jax.ShapeDtypeStruct((n, 4), jnp.float32),
            jax.ShapeDtypeStruct((1, HID), jnp.float32),
        ],
        scratch_shapes=[
            pltpu.VMEM((1, HID), jnp.float32),
            pltpu.VMEM((bn, HID), jnp.float32),
        ],
    )(gi, w_hh_t, b_hh_n, fc_W, fc_b, h0)


def kernel(x, edge_index, hidden_state, gcn_W, gcn_b, W_ih, W_hh, b_ih, b_hh,
           fc_W, fc_b):
    n = x.shape[0]
    e = edge_index.shape[1]
    np_ = 50176
    e_pad = 819200

    src = edge_index[0].astype(jnp.int32)
    dst = edge_index[1].astype(jnp.int32)
    padi = jnp.full((e_pad - e,), n, jnp.int32)
    src_pad = jnp.concatenate([src, padi])
    dst_pad = jnp.concatenate([dst, padi])
    srcx = jnp.concatenate([src_pad, src_pad + np_]).reshape(-1, 128)
    dst2 = dst_pad.reshape(-1, 128)

    zeros1 = jnp.zeros((np_,), jnp.float32)
    zrows = jnp.zeros((np_, 32), jnp.float32)

    degp = _sc_deg(dst_pad, zeros1, np_, e_pad)

    xp = jnp.zeros((np_, 7), x.dtype).at[:n].set(x)
    h, hp, dinvc = _tc_prep_a(degp, xp, gcn_W, np_)

    hpc = jnp.concatenate([hp[:, :32], hp[:, 32:]], axis=0)
    aggp = _sc_agg(srcx, dst2, hpc, zrows, np_, e_pad)

    w_ih_t = W_ih.T
    bias_g = (b_ih + jnp.concatenate(
        [b_hh[:2 * HID], jnp.zeros((HID,), jnp.float32)])).reshape(1, G3)
    gi = _tc_prep_b(aggp[:, :n, :], dinvc[:n], h[:n], gcn_b.reshape(1, HID),
                    w_ih_t, bias_g, n, np_)

    w_hh_t = W_hh.T
    b_hh_n = jnp.concatenate(
        [jnp.zeros((2 * HID,), jnp.float32), b_hh[2 * HID:]]).reshape(1, G3)
    h0 = hidden_state[0]
    y, hlast = _tc_gru(gi, w_hh_t, b_hh_n, fc_W, fc_b.reshape(1, 4), h0, n)

    new_x = jnp.concatenate([x[:, :3], y], axis=1)
    return new_x, hlast[None]
